# Initial kernel scaffold; baseline (speedup 1.0000x reference)
import jax
import jax.numpy as jnp
from jax import lax
from jax.experimental import pallas as pl
from jax.experimental.pallas import tpu as pltpu
from jax.experimental.pallas import tpu_sc as plsc

N = 50000
E = 800000
G = 500
D = 80
H = 40
V = 10000

NC = 2
NS = 16
NW = NC * NS

DEG_STRIPE = 3136
DEG_PAD = NS * DEG_STRIPE
ACC_STRIPE = 3128
ACC_PAD = NS * ACC_STRIPE

ECHUNK = 1000

ROWS_W = 1568
ROWCHUNK = 224

_MESH = plsc.VectorSubcoreMesh(core_axis_name="c", subcore_axis_name="s")


def _worker_base(w):
    return (N * w // NW) // 8 * 8


def _emb_body(x_hbm, emb_hbm, h0_hbm, emb_v, xc_v, out_v):
    c = lax.axis_index("c")
    s = lax.axis_index("s")
    w = s * NC + c
    pltpu.sync_copy(emb_hbm, emb_v)
    base = _worker_base(w)
    io = lax.iota(jnp.int32, 16)
    half = lax.shift_right_logical(io, 1)
    par = lax.bitwise_and(io, 1)
    cols = [half + (c5 * 8) for c5 in range(5)]

    @pl.loop(0, ROWS_W // ROWCHUNK)
    def _(j):
        row0 = base + j * ROWCHUNK
        pltpu.sync_copy(x_hbm.at[pl.ds(row0, ROWCHUNK)], xc_v)

        @pl.loop(0, ROWCHUNK)
        def _(r):
            rs = jnp.full((16,), r, jnp.int32)
            for c5 in range(5):
                xv = plsc.load_gather(xc_v, [rs, cols[c5]])
                val = plsc.load_gather(emb_v, [xv, par])
                out_v[r, pl.ds(c5 * 16, 16)] = val

        pltpu.sync_copy(out_v, h0_hbm.at[pl.ds(row0, ROWCHUNK)])


def _emb_lookup(x, emb):
    return pl.kernel(
        _emb_body,
        out_type=jax.ShapeDtypeStruct((N, D), jnp.float32),
        mesh=_MESH,
        scratch_types=[
            pltpu.VMEM((V, 2), jnp.float32),
            pltpu.VMEM((ROWCHUNK, H), jnp.int32),
            pltpu.VMEM((ROWCHUNK, D), jnp.float32),
        ],
    )(x, emb)


def _deg_body(ei_hbm, degp_hbm, acc_sh, ones_v, idx_v, zb_v):
    c = lax.axis_index("c")
    s = lax.axis_index("s")
    zero16 = jnp.zeros((16,), jnp.float32)
    one16 = jnp.ones((16,), jnp.float32)

    @pl.loop(0, DEG_STRIPE, step=16)
    def _(i):
        zb_v[pl.ds(i, 16)] = zero16

    @pl.loop(0, ECHUNK - 8, step=16)
    def _(i):
        ones_v[pl.ds(i, 16)] = one16

    ones_v[pl.ds(ECHUNK - 16, 16)] = one16

    pltpu.sync_copy(zb_v, acc_sh.at[pl.ds(s * DEG_STRIPE, DEG_STRIPE)])
    plsc.subcore_barrier()

    epw = E // NC // NS

    @pl.loop(0, epw // ECHUNK)
    def _(k):
        off = c * (E // NC) + s * epw + k * ECHUNK
        pltpu.sync_copy(ei_hbm.at[1, pl.ds(off, ECHUNK)], idx_v)
        pltpu.sync_copy(ones_v, acc_sh.at[idx_v], add=True)

    plsc.subcore_barrier()
    pltpu.sync_copy(acc_sh.at[pl.ds(s * DEG_STRIPE, DEG_STRIPE)],
                    degp_hbm.at[c, pl.ds(s * DEG_STRIPE, DEG_STRIPE)])


def _degrees(edge_index):
    return pl.kernel(
        _deg_body,
        out_type=jax.ShapeDtypeStruct((NC, DEG_PAD), jnp.float32),
        mesh=_MESH,
        scratch_types=[
            pltpu.VMEM_SHARED((DEG_PAD,), jnp.float32),
            pltpu.VMEM((ECHUNK,), jnp.float32),
            pltpu.VMEM((ECHUNK,), jnp.int32),
            pltpu.VMEM((DEG_STRIPE,), jnp.float32),
        ],
    )(edge_index)


def _scat_body(ei_hbm, sl_hbm, sr_hbm, acc_hbm, acc_sh, src_v, dst_v, rows_v):
    c = lax.axis_index("c")
    s = lax.axis_index("s")
    zero16 = jnp.zeros((16,), jnp.float32)

    @pl.loop(0, ECHUNK)
    def _(r):
        for c5 in range(H // 16):
            rows_v[r, pl.ds(c5 * 16, 16)] = zero16
        rows_v[r, pl.ds(H - 16, 16)] = zero16

    for part in range(3):
        pltpu.sync_copy(rows_v, acc_sh.at[pl.ds(s * ACC_STRIPE + part * ECHUNK, ECHUNK)])
    pltpu.sync_copy(rows_v.at[pl.ds(0, ACC_STRIPE - 3 * ECHUNK)],
                    acc_sh.at[pl.ds(s * ACC_STRIPE + 3 * ECHUNK, ACC_STRIPE - 3 * ECHUNK)])
    plsc.subcore_barrier()

    epw = E // NS

    @pl.loop(0, epw // ECHUNK)
    def _(k):
        off = s * epw + k * ECHUNK
        pltpu.sync_copy(ei_hbm.at[0, pl.ds(off, ECHUNK)], src_v)
        pltpu.sync_copy(ei_hbm.at[1, pl.ds(off, ECHUNK)], dst_v)

        @pl.when(c == 0)
        def _():
            pltpu.sync_copy(sl_hbm.at[src_v], rows_v)

        @pl.when(c == 1)
        def _():
            pltpu.sync_copy(sr_hbm.at[src_v], rows_v)

        pltpu.sync_copy(rows_v, acc_sh.at[dst_v], add=True)

    plsc.subcore_barrier()
    pltpu.sync_copy(acc_sh.at[pl.ds(s * ACC_STRIPE, ACC_STRIPE)],
                    acc_hbm.at[c, pl.ds(s * ACC_STRIPE, ACC_STRIPE)])


def _aggregate(edge_index, sl, sr):
    return pl.kernel(
        _scat_body,
        out_type=jax.ShapeDtypeStruct((NC, ACC_PAD, H), jnp.float32),
        mesh=_MESH,
        scratch_types=[
            pltpu.VMEM_SHARED((ACC_PAD, H), jnp.float32),
            pltpu.VMEM((ECHUNK,), jnp.int32),
            pltpu.VMEM((ECHUNK,), jnp.int32),
            pltpu.VMEM((ECHUNK, H), jnp.float32),
        ],
    )(edge_index, sl, sr)


def _segmax_body(pre_hbm, coef_hbm, batch_hbm, part_hbm, acc_v, hm_v, bt_v, coef_v):
    c = lax.axis_index("c")
    s = lax.axis_index("s")
    w = s * NC + c
    pltpu.sync_copy(coef_hbm, coef_v)
    neg = jnp.full((16,), -jnp.inf, jnp.float32)
    io = lax.iota(jnp.int32, 16)

    @pl.loop(0, G * D, step=16)
    def _(i):
        acc_v[pl.ds(i, 16)] = neg

    base = _worker_base(w)

    @pl.loop(0, ROWS_W // ROWCHUNK)
    def _(j):
        row0 = base + j * ROWCHUNK
        pltpu.sync_copy(pre_hbm.at[pl.ds(row0, ROWCHUNK)], hm_v)
        pltpu.sync_copy(batch_hbm.at[pl.ds(row0, ROWCHUNK)], bt_v)

        @pl.loop(0, ROWCHUNK)
        def _(r):
            rs = jnp.full((16,), r, jnp.int32)
            gb = plsc.load_gather(bt_v, [rs]) * D
            for c5 in range(5):
                a = coef_v[0, pl.ds(c5 * 16, 16)]
                b = coef_v[1, pl.ds(c5 * 16, 16)]
                xv = hm_v[r, pl.ds(c5 * 16, 16)]
                val = jnp.maximum(xv * a + b, 0.0)
                idx = gb + (io + c5 * 16)
                cur = plsc.load_gather(acc_v, [idx])
                plsc.store_scatter(acc_v, [idx], jnp.maximum(cur, val))

    pltpu.sync_copy(acc_v, part_hbm.at[w])


def _segmax(pre2, coef2, batch):
    return pl.kernel(
        _segmax_body,
        out_type=jax.ShapeDtypeStruct((NW, G * D), jnp.float32),
        mesh=_MESH,
        scratch_types=[
            pltpu.VMEM((G * D,), jnp.float32),
            pltpu.VMEM((ROWCHUNK, D), jnp.float32),
            pltpu.VMEM((ROWCHUNK,), jnp.int32),
            pltpu.VMEM((2, D), jnp.float32),
        ],
    )(pre2, coef2, batch)


TB = 2000
NGRID = N // TB


def _prep1_body(h0_ref, degp_ref, w_ref, sl_ref, sr_ref, dinv_ref):
    deg = degp_ref[0, :] + degp_ref[1, :] + 1.0
    dv = lax.rsqrt(deg)
    hw = jnp.dot(h0_ref[...], w_ref[...], preferred_element_type=jnp.float32)
    sc = dv[:, None] * hw
    sl_ref[...] = sc[:, :H]
    sr_ref[...] = sc[:, H:]
    dinv_ref[...] = dv


def _prep1(h0, degp, W1):
    return pl.pallas_call(
        _prep1_body,
        grid=(NGRID,),
        in_specs=[
            pl.BlockSpec((TB, D), lambda i: (i, 0)),
            pl.BlockSpec((NC, TB), lambda i: (0, i)),
            pl.BlockSpec((D, D), lambda i: (0, 0)),
        ],
        out_specs=[
            pl.BlockSpec((TB, H), lambda i: (i, 0)),
            pl.BlockSpec((TB, H), lambda i: (i, 0)),
            pl.BlockSpec((TB,), lambda i: (i,)),
        ],
        out_shape=[
            jax.ShapeDtypeStruct((N, H), jnp.float32),
            jax.ShapeDtypeStruct((N, H), jnp.float32),
            jax.ShapeDtypeStruct((N,), jnp.float32),
        ],
    )(h0, degp, W1)


def _post_body(acc_ref, sl_ref, sr_ref, dinv_ref, b_ref, g_ref, be_ref,
               pre_ref, coef_ref, stat_ref):
    i = pl.program_id(0)
    dv = dinv_ref[...]
    pre = dv[:, None] * jnp.concatenate(
        [acc_ref[0] + sl_ref[...], acc_ref[1] + sr_ref[...]], axis=1) + b_ref[...]
    pre_ref[...] = pre
    ps = jnp.sum(pre, axis=0)
    pq = jnp.sum(pre * pre, axis=0)

    @pl.when(i == 0)
    def _():
        stat_ref[0, :] = ps
        stat_ref[1, :] = pq

    @pl.when(i > 0)
    def _():
        stat_ref[0, :] += ps
        stat_ref[1, :] += pq

    @pl.when(i == NGRID - 1)
    def _():
        mu = stat_ref[0, :] * (1.0 / N)
        var = stat_ref[1, :] * (1.0 / N) - mu * mu
        a = g_ref[...] * lax.rsqrt(var + 1e-5)
        coef_ref[0, :] = a
        coef_ref[1, :] = be_ref[...] - mu * a


def _post(acc, sl, sr, dinv, b, g, be):
    return pl.pallas_call(
        _post_body,
        grid=(NGRID,),
        in_specs=[
            pl.BlockSpec((NC, TB, H), lambda i: (0, i, 0)),
            pl.BlockSpec((TB, H), lambda i: (i, 0)),
            pl.BlockSpec((TB, H), lambda i: (i, 0)),
            pl.BlockSpec((TB,), lambda i: (i,)),
            pl.BlockSpec((D,), lambda i: (0,)),
            pl.BlockSpec((D,), lambda i: (0,)),
            pl.BlockSpec((D,), lambda i: (0,)),
        ],
        out_specs=[
            pl.BlockSpec((TB, D), lambda i: (i, 0)),
            pl.BlockSpec((2, D), lambda i: (0, 0)),
        ],
        out_shape=[
            jax.ShapeDtypeStruct((N, D), jnp.float32),
            jax.ShapeDtypeStruct((2, D), jnp.float32),
        ],
        scratch_shapes=[pltpu.VMEM((2, D), jnp.float32)],
    )(acc, sl, sr, dinv, b, g, be)


def _prep2_body(pre_ref, coef_ref, w_ref, dinv_ref, sl_ref, sr_ref):
    h1 = jnp.maximum(pre_ref[...] * coef_ref[0, :] + coef_ref[1, :], 0.0)
    hw = jnp.dot(h1, w_ref[...], preferred_element_type=jnp.float32)
    sc = dinv_ref[...][:, None] * hw
    sl_ref[...] = sc[:, :H]
    sr_ref[...] = sc[:, H:]


def _prep2(pre1, coef1, W2, dinv):
    return pl.pallas_call(
        _prep2_body,
        grid=(NGRID,),
        in_specs=[
            pl.BlockSpec((TB, D), lambda i: (i, 0)),
            pl.BlockSpec((2, D), lambda i: (0, 0)),
            pl.BlockSpec((D, D), lambda i: (0, 0)),
            pl.BlockSpec((TB,), lambda i: (i,)),
        ],
        out_specs=[
            pl.BlockSpec((TB, H), lambda i: (i, 0)),
            pl.BlockSpec((TB, H), lambda i: (i, 0)),
        ],
        out_shape=[
            jax.ShapeDtypeStruct((N, H), jnp.float32),
            jax.ShapeDtypeStruct((N, H), jnp.float32),
        ],
    )(pre1, coef1, W2, dinv)


def _head_body(part_ref, lw_ref, lb_ref, out_ref):
    p = part_ref[...].reshape(NW, G, D)
    pooled = jnp.max(p, axis=0)
    out_ref[...] = jnp.dot(pooled, lw_ref[...],
                           preferred_element_type=jnp.float32) + lb_ref[...]


def _head(part, linW, linb):
    return pl.pallas_call(
        _head_body,
        out_shape=jax.ShapeDtypeStruct((G, 20), jnp.float32),
    )(part, linW, linb)


def kernel(x, edge_index, batch, emb, W1, b1, g1, be1, W2, b2, g2, be2, linW, linb):
    h0 = _emb_lookup(x, emb)
    degp = _degrees(edge_index)
    sl1, sr1, dinv = _prep1(h0, degp, W1)
    acc1 = _aggregate(edge_index, sl1, sr1)
    pre1, coef1 = _post(acc1, sl1, sr1, dinv, b1, g1, be1)
    sl2, sr2 = _prep2(pre1, coef1, W2, dinv)
    acc2 = _aggregate(edge_index, sl2, sr2)
    pre2, coef2 = _post(acc2, sl2, sr2, dinv, b2, g2, be2)
    part = _segmax(pre2, coef2, batch)
    return _head(part, linW, linb)


# speedup vs baseline: 14.9283x; 14.9283x over previous
import jax
import jax.numpy as jnp
from jax import lax
from jax.experimental import pallas as pl
from jax.experimental.pallas import tpu as pltpu
from jax.experimental.pallas import tpu_sc as plsc

N = 50000
E = 800000
G = 500
D = 80
H = 40
V = 10000

NC = 2
NS = 16
NW = NC * NS

DEG_STRIPE = 3136
DEG_PAD = NS * DEG_STRIPE
ACC_STRIPE = 3128
ACC_PAD = NS * ACC_STRIPE

ECHUNK = 1000

ROWS_W = 1568
ROWCHUNK = 224

_MESH = plsc.VectorSubcoreMesh(core_axis_name="c", subcore_axis_name="s")

_CP = pltpu.CompilerParams(needs_layout_passes=False, use_tc_tiling_on_sc=False)


def _worker_base(w):
    return (N * w // NW) // 8 * 8


def _emb_body(x_hbm, emb_hbm, h0_hbm, emb_v, xc_v, out_v):
    c = lax.axis_index("c")
    s = lax.axis_index("s")
    w = s * NC + c
    pltpu.sync_copy(emb_hbm, emb_v)
    base = _worker_base(w)
    io = lax.iota(jnp.int32, 16)
    half = lax.shift_right_logical(io, 1)
    par = lax.bitwise_and(io, 1)
    cols = [half + (c5 * 8) for c5 in range(5)]

    @pl.loop(0, ROWS_W // ROWCHUNK)
    def _(j):
        row0 = base + j * ROWCHUNK
        pltpu.sync_copy(x_hbm.at[pl.ds(row0, ROWCHUNK)], xc_v)

        @pl.loop(0, ROWCHUNK)
        def _(r):
            rs = jnp.full((16,), r, jnp.int32)
            for c5 in range(5):
                xv = plsc.load_gather(xc_v, [rs, cols[c5]])
                val = plsc.load_gather(emb_v, [xv + xv + par])
                out_v[r, pl.ds(c5 * 16, 16)] = val

        pltpu.sync_copy(out_v, h0_hbm.at[pl.ds(row0, ROWCHUNK)])


def _emb_lookup(x, emb):
    return pl.kernel(
        _emb_body,
        out_type=jax.ShapeDtypeStruct((N, D), jnp.float32),
        mesh=_MESH,
        compiler_params=_CP,
        scratch_types=[
            pltpu.VMEM((2 * V,), jnp.float32),
            pltpu.VMEM((ROWCHUNK, H), jnp.int32),
            pltpu.VMEM((ROWCHUNK, D), jnp.float32),
        ],
    )(x, emb.reshape(2 * V))


def _deg_body(ei_hbm, degp_hbm, acc_sh, ones_v, idx_v, zb_v):
    c = lax.axis_index("c")
    s = lax.axis_index("s")
    zero16 = jnp.zeros((16,), jnp.float32)
    one16 = jnp.ones((16,), jnp.float32)

    @pl.loop(0, DEG_STRIPE, step=16)
    def _(i):
        zb_v[pl.ds(i, 16)] = zero16

    @pl.loop(0, ECHUNK - 8, step=16)
    def _(i):
        ones_v[pl.ds(i, 16)] = one16

    ones_v[pl.ds(ECHUNK - 16, 16)] = one16

    pltpu.sync_copy(zb_v, acc_sh.at[pl.ds(s * DEG_STRIPE, DEG_STRIPE)])
    plsc.subcore_barrier()

    epw = E // NC // NS

    @pl.loop(0, epw // ECHUNK)
    def _(k):
        off = c * (E // NC) + s * epw + k * ECHUNK
        pltpu.sync_copy(ei_hbm.at[pl.ds(E + off, ECHUNK)], idx_v)
        pltpu.sync_copy(ones_v, acc_sh.at[idx_v], add=True)

    plsc.subcore_barrier()
    pltpu.sync_copy(acc_sh.at[pl.ds(s * DEG_STRIPE, DEG_STRIPE)], zb_v)
    pltpu.sync_copy(zb_v, degp_hbm.at[pl.ds(c * DEG_PAD + s * DEG_STRIPE, DEG_STRIPE)])


def _degrees(edge_index):
    return pl.kernel(
        _deg_body,
        out_type=jax.ShapeDtypeStruct((NC * DEG_PAD,), jnp.float32),
        mesh=_MESH,
        compiler_params=_CP,
        scratch_types=[
            pltpu.VMEM_SHARED((DEG_PAD,), jnp.float32),
            pltpu.VMEM((ECHUNK,), jnp.float32),
            pltpu.VMEM((ECHUNK,), jnp.int32),
            pltpu.VMEM((DEG_STRIPE,), jnp.float32),
        ],
    )(edge_index.reshape(2 * E))


SL = 16
NSLOT = 6


def _scat_body(ei_hbm, s0, s1, s2, s3, s4, acc_hbm, acc_sh, src_v, dst_v, rows_v):
    c = lax.axis_index("c")
    s = lax.axis_index("s")
    zero16 = jnp.zeros((16,), jnp.float32)

    def zero_rows():
        @pl.loop(0, ECHUNK)
        def _(r):
            rows_v[r, :] = zero16

    def one_pass(s_hbm, slot, elo, nedge):
        zero_rows()
        for part in range(3):
            pltpu.sync_copy(rows_v, acc_sh.at[pl.ds(s * ACC_STRIPE + part * ECHUNK, ECHUNK)])
        tail = ACC_STRIPE - 3 * ECHUNK
        pltpu.sync_copy(rows_v.at[pl.ds(0, tail)],
                        acc_sh.at[pl.ds(s * ACC_STRIPE + 3 * ECHUNK, tail)])
        plsc.subcore_barrier()

        epw = nedge // NS

        @pl.loop(0, epw // ECHUNK)
        def _(k):
            off = elo + s * epw + k * ECHUNK
            pltpu.sync_copy(ei_hbm.at[pl.ds(off, ECHUNK)], src_v)
            pltpu.sync_copy(ei_hbm.at[pl.ds(E + off, ECHUNK)], dst_v)
            pltpu.sync_copy(s_hbm.at[src_v], rows_v)
            pltpu.sync_copy(rows_v, acc_sh.at[dst_v], add=True)

        plsc.subcore_barrier()
        for part in range(3):
            pltpu.sync_copy(acc_sh.at[pl.ds(s * ACC_STRIPE + part * ECHUNK, ECHUNK)], rows_v)
            pltpu.sync_copy(
                rows_v,
                acc_hbm.at[pl.ds(slot * ACC_PAD + s * ACC_STRIPE + part * ECHUNK, ECHUNK)])
        pltpu.sync_copy(acc_sh.at[pl.ds(s * ACC_STRIPE + 3 * ECHUNK, tail)],
                        rows_v.at[pl.ds(0, tail)])
        pltpu.sync_copy(
            rows_v.at[pl.ds(0, tail)],
            acc_hbm.at[pl.ds(slot * ACC_PAD + s * ACC_STRIPE + 3 * ECHUNK, tail)])

    @pl.when(c == 0)
    def _():
        one_pass(s0, 0, 0, E)
        one_pass(s1, 1, 0, E)
        one_pass(s4, 4, 0, E // 2)

    @pl.when(c == 1)
    def _():
        one_pass(s2, 2, 0, E)
        one_pass(s3, 3, 0, E)
        one_pass(s4, 5, E // 2, E // 2)


def _aggregate(edge_index, slices):
    return pl.kernel(
        _scat_body,
        out_type=jax.ShapeDtypeStruct((NSLOT * ACC_PAD, SL), jnp.float32),
        mesh=_MESH,
        compiler_params=_CP,
        scratch_types=[
            pltpu.VMEM_SHARED((ACC_PAD, SL), jnp.float32),
            pltpu.VMEM((ECHUNK,), jnp.int32),
            pltpu.VMEM((ECHUNK,), jnp.int32),
            pltpu.VMEM((ECHUNK, SL), jnp.float32),
        ],
    )(edge_index.reshape(2 * E), *slices)


def _segmax_body(pre_hbm, coef_hbm, batch_hbm, part_hbm, acc_v, hm_v, bt_v, coef_v):
    c = lax.axis_index("c")
    s = lax.axis_index("s")
    w = s * NC + c
    pltpu.sync_copy(coef_hbm, coef_v)
    neg = jnp.full((16,), -jnp.inf, jnp.float32)
    io = lax.iota(jnp.int32, 16)

    @pl.loop(0, G * D, step=16)
    def _(i):
        acc_v[pl.ds(i, 16)] = neg

    base = _worker_base(w)

    @pl.loop(0, ROWS_W // ROWCHUNK)
    def _(j):
        row0 = base + j * ROWCHUNK
        pltpu.sync_copy(pre_hbm.at[pl.ds(row0, ROWCHUNK)], hm_v)
        pltpu.sync_copy(batch_hbm.at[pl.ds(row0, ROWCHUNK)], bt_v)

        @pl.loop(0, ROWCHUNK)
        def _(r):
            rs = jnp.full((16,), r, jnp.int32)
            gb = plsc.load_gather(bt_v, [rs]) * D
            for c5 in range(5):
                a = coef_v[0, pl.ds(c5 * 16, 16)]
                b = coef_v[1, pl.ds(c5 * 16, 16)]
                xv = hm_v[r, pl.ds(c5 * 16, 16)]
                val = jnp.maximum(xv * a + b, 0.0)
                idx = gb + (io + c5 * 16)
                cur = plsc.load_gather(acc_v, [idx])
                plsc.store_scatter(acc_v, [idx], jnp.maximum(cur, val))

    pltpu.sync_copy(acc_v, part_hbm.at[pl.ds(w * (G * D), G * D)])


def _segmax(pre2, coef2, batch):
    return pl.kernel(
        _segmax_body,
        out_type=jax.ShapeDtypeStruct((NW * G * D,), jnp.float32),
        mesh=_MESH,
        compiler_params=_CP,
        scratch_types=[
            pltpu.VMEM((G * D,), jnp.float32),
            pltpu.VMEM((ROWCHUNK, D), jnp.float32),
            pltpu.VMEM((ROWCHUNK,), jnp.int32),
            pltpu.VMEM((2, D), jnp.float32),
        ],
    )(pre2, coef2, batch)


TB = 2000
NGRID = N // TB


def _slice_outs():
    return ([pl.BlockSpec((TB, SL), lambda i: (i, 0)) for _ in range(5)],
            [jax.ShapeDtypeStruct((N, SL), jnp.float32) for _ in range(5)])


def _prep1_body(h0_ref, degp_ref, w_ref, *out_refs):
    deg = degp_ref[0] + degp_ref[1] + 1.0
    dv = lax.rsqrt(deg)
    hw = jnp.dot(h0_ref[...], w_ref[...], preferred_element_type=jnp.float32)
    sc = dv * hw
    for k in range(5):
        out_refs[k][...] = sc[:, k * SL:(k + 1) * SL]
    out_refs[5][...] = dv


def _prep1(h0, degp, W1):
    sspecs, sshapes = _slice_outs()
    return pl.pallas_call(
        _prep1_body,
        grid=(NGRID,),
        in_specs=[
            pl.BlockSpec((TB, D), lambda i: (i, 0)),
            pl.BlockSpec((NC, TB, 1), lambda i: (0, i, 0)),
            pl.BlockSpec((D, D), lambda i: (0, 0)),
        ],
        out_specs=sspecs + [pl.BlockSpec((TB, 1), lambda i: (i, 0))],
        out_shape=sshapes + [jax.ShapeDtypeStruct((N, 1), jnp.float32)],
    )(h0, degp.reshape(NC, DEG_PAD, 1), W1)


def _post_body(acc_ref, s0, s1, s2, s3, s4, dinv_ref, b_ref, g_ref, be_ref,
               pre_ref, coef_ref, stat_ref):
    i = pl.program_id(0)
    dv = dinv_ref[...]
    agg = jnp.concatenate(
        [acc_ref[0] + s0[...], acc_ref[1] + s1[...], acc_ref[2] + s2[...],
         acc_ref[3] + s3[...], acc_ref[4] + acc_ref[5] + s4[...]], axis=1)
    pre = dv * agg + b_ref[...]
    pre_ref[...] = pre
    ps = jnp.sum(pre, axis=0)
    pq = jnp.sum(pre * pre, axis=0)

    @pl.when(i == 0)
    def _():
        stat_ref[0, :] = ps
        stat_ref[1, :] = pq

    @pl.when(i > 0)
    def _():
        stat_ref[0, :] += ps
        stat_ref[1, :] += pq

    @pl.when(i == NGRID - 1)
    def _():
        mu = stat_ref[0, :] * (1.0 / N)
        var = stat_ref[1, :] * (1.0 / N) - mu * mu
        a = g_ref[...] * lax.rsqrt(var + 1e-5)
        coef_ref[0, :] = a
        coef_ref[1, :] = be_ref[...] - mu * a


def _post(acc, slices, dinv, b, g, be):
    return pl.pallas_call(
        _post_body,
        grid=(NGRID,),
        in_specs=[pl.BlockSpec((NSLOT, TB, SL), lambda i: (0, i, 0))]
        + [pl.BlockSpec((TB, SL), lambda i: (i, 0)) for _ in range(5)]
        + [
            pl.BlockSpec((TB, 1), lambda i: (i, 0)),
            pl.BlockSpec((D,), lambda i: (0,)),
            pl.BlockSpec((D,), lambda i: (0,)),
            pl.BlockSpec((D,), lambda i: (0,)),
        ],
        out_specs=[
            pl.BlockSpec((TB, D), lambda i: (i, 0)),
            pl.BlockSpec((2, D), lambda i: (0, 0)),
        ],
        out_shape=[
            jax.ShapeDtypeStruct((N, D), jnp.float32),
            jax.ShapeDtypeStruct((2, D), jnp.float32),
        ],
        scratch_shapes=[pltpu.VMEM((2, D), jnp.float32)],
    )(acc, *slices, dinv, b, g, be)


def _prep2_body(pre_ref, coef_ref, w_ref, dinv_ref, *out_refs):
    h1 = jnp.maximum(pre_ref[...] * coef_ref[0, :] + coef_ref[1, :], 0.0)
    hw = jnp.dot(h1, w_ref[...], preferred_element_type=jnp.float32)
    sc = dinv_ref[...] * hw
    for k in range(5):
        out_refs[k][...] = sc[:, k * SL:(k + 1) * SL]


def _prep2(pre1, coef1, W2, dinv):
    sspecs, sshapes = _slice_outs()
    return pl.pallas_call(
        _prep2_body,
        grid=(NGRID,),
        in_specs=[
            pl.BlockSpec((TB, D), lambda i: (i, 0)),
            pl.BlockSpec((2, D), lambda i: (0, 0)),
            pl.BlockSpec((D, D), lambda i: (0, 0)),
            pl.BlockSpec((TB, 1), lambda i: (i, 0)),
        ],
        out_specs=sspecs,
        out_shape=sshapes,
    )(pre1, coef1, W2, dinv)


def _head_body(part_ref, lw_ref, lb_ref, out_ref):
    p = part_ref[...].reshape(NW, G, D)
    pooled = jnp.max(p, axis=0)
    out_ref[...] = jnp.dot(pooled, lw_ref[...],
                           preferred_element_type=jnp.float32) + lb_ref[...]


def _head(part, linW, linb):
    return pl.pallas_call(
        _head_body,
        out_shape=jax.ShapeDtypeStruct((G, 20), jnp.float32),
    )(part, linW, linb)


def kernel(x, edge_index, batch, emb, W1, b1, g1, be1, W2, b2, g2, be2, linW, linb):
    h0 = _emb_lookup(x, emb)
    degp = _degrees(edge_index)
    *sl1, dinv = _prep1(h0, degp, W1)
    acc1 = _aggregate(edge_index, sl1).reshape(NSLOT, ACC_PAD, SL)
    pre1, coef1 = _post(acc1, sl1, dinv, b1, g1, be1)
    sl2 = _prep2(pre1, coef1, W2, dinv)
    acc2 = _aggregate(edge_index, sl2).reshape(NSLOT, ACC_PAD, SL)
    pre2, coef2 = _post(acc2, sl2, dinv, b2, g2, be2)
    part = _segmax(pre2, coef2, batch)
    return _head(part.reshape(NW, G * D), linW, linb)


# speedup vs baseline: 18.8738x; 1.2643x over previous
import jax
import jax.numpy as jnp
from jax import lax
from jax.experimental import pallas as pl
from jax.experimental.pallas import tpu as pltpu
from jax.experimental.pallas import tpu_sc as plsc

N = 50000
E = 800000
G = 500
D = 80
H = 40
V = 10000

NC = 2
NS = 16
NW = NC * NS

DEG_STRIPE = 3136
DEG_PAD = NS * DEG_STRIPE
ACC_STRIPE = 3128
ACC_PAD = NS * ACC_STRIPE

ECHUNK = 1000

ROWS_W = 1568
ROWCHUNK = 224

_MESH = plsc.VectorSubcoreMesh(core_axis_name="c", subcore_axis_name="s")

_CP = pltpu.CompilerParams(needs_layout_passes=False, use_tc_tiling_on_sc=False)


def _worker_base(w):
    return (N * w // NW) // 8 * 8


def _emb_body(x_hbm, emb_hbm, h0_hbm, emb_v, xc_v, out_v):
    c = lax.axis_index("c")
    s = lax.axis_index("s")
    w = s * NC + c
    pltpu.sync_copy(emb_hbm, emb_v)
    base = _worker_base(w)
    io = lax.iota(jnp.int32, 16)
    half = lax.shift_right_logical(io, 1)
    par = lax.bitwise_and(io, 1)
    cols = [half + (c5 * 8) for c5 in range(5)]

    @pl.loop(0, ROWS_W // ROWCHUNK)
    def _(j):
        row0 = base + j * ROWCHUNK
        pltpu.sync_copy(x_hbm.at[pl.ds(row0, ROWCHUNK)], xc_v)

        @pl.loop(0, ROWCHUNK)
        def _(r):
            rs = jnp.full((16,), r, jnp.int32)
            for c5 in range(5):
                xv = plsc.load_gather(xc_v, [rs, cols[c5]])
                val = plsc.load_gather(emb_v, [xv + xv + par])
                out_v[r, pl.ds(c5 * 16, 16)] = val

        pltpu.sync_copy(out_v, h0_hbm.at[pl.ds(row0, ROWCHUNK)])


def _emb_lookup(x, emb):
    return pl.kernel(
        _emb_body,
        out_type=jax.ShapeDtypeStruct((N, D), jnp.float32),
        mesh=_MESH,
        compiler_params=_CP,
        scratch_types=[
            pltpu.VMEM((2 * V,), jnp.float32),
            pltpu.VMEM((ROWCHUNK, H), jnp.int32),
            pltpu.VMEM((ROWCHUNK, D), jnp.float32),
        ],
    )(x, emb.reshape(2 * V))


def _deg_body(ei_hbm, degp_hbm, acc_sh, ones_v, idx_v, zb_v):
    c = lax.axis_index("c")
    s = lax.axis_index("s")
    zero16 = jnp.zeros((16,), jnp.float32)
    one16 = jnp.ones((16,), jnp.float32)

    @pl.loop(0, DEG_STRIPE, step=16)
    def _(i):
        zb_v[pl.ds(i, 16)] = zero16

    @pl.loop(0, ECHUNK - 8, step=16)
    def _(i):
        ones_v[pl.ds(i, 16)] = one16

    ones_v[pl.ds(ECHUNK - 16, 16)] = one16

    pltpu.sync_copy(zb_v, acc_sh.at[pl.ds(s * DEG_STRIPE, DEG_STRIPE)])
    plsc.subcore_barrier()

    epw = E // NC // NS

    @pl.loop(0, epw // ECHUNK)
    def _(k):
        off = c * (E // NC) + s * epw + k * ECHUNK
        pltpu.sync_copy(ei_hbm.at[pl.ds(E + off, ECHUNK)], idx_v)
        pltpu.sync_copy(ones_v, acc_sh.at[idx_v], add=True)

    plsc.subcore_barrier()
    pltpu.sync_copy(acc_sh.at[pl.ds(s * DEG_STRIPE, DEG_STRIPE)], zb_v)
    pltpu.sync_copy(zb_v, degp_hbm.at[pl.ds(c * DEG_PAD + s * DEG_STRIPE, DEG_STRIPE)])


def _degrees(edge_index):
    return pl.kernel(
        _deg_body,
        out_type=jax.ShapeDtypeStruct((NC * DEG_PAD,), jnp.float32),
        mesh=_MESH,
        compiler_params=_CP,
        scratch_types=[
            pltpu.VMEM_SHARED((DEG_PAD,), jnp.float32),
            pltpu.VMEM((ECHUNK,), jnp.float32),
            pltpu.VMEM((ECHUNK,), jnp.int32),
            pltpu.VMEM((DEG_STRIPE,), jnp.float32),
        ],
    )(edge_index.reshape(2 * E))


SL = 16
NSLOT = 6


def _scat_body(ei_hbm, s0, s1, s2, s3, s4, acc_hbm, acc_sh,
               src0, dst0, src1, dst1, rows0, rows1, zb_v, semA, semB):
    c = lax.axis_index("c")
    s = lax.axis_index("s")
    zero16 = jnp.zeros((16,), jnp.float32)

    @pl.loop(0, ECHUNK)
    def _(r):
        zb_v[r, :] = zero16

    bufs = ((src0, dst0, rows0, semA), (src1, dst1, rows1, semB))

    def one_pass(s_hbm, slot, elo, nedge):
        for part in range(3):
            pltpu.sync_copy(zb_v, acc_sh.at[pl.ds(s * ACC_STRIPE + part * ECHUNK, ECHUNK)])
        tail = ACC_STRIPE - 3 * ECHUNK
        pltpu.sync_copy(zb_v.at[pl.ds(0, tail)],
                        acc_sh.at[pl.ds(s * ACC_STRIPE + 3 * ECHUNK, tail)])
        plsc.subcore_barrier()

        epw = nedge // NS
        nch = epw // ECHUNK
        base = elo + s * epw

        def prefetch(k, sv, dv, rv, sem):
            off = base + k * ECHUNK
            pltpu.sync_copy(ei_hbm.at[pl.ds(off, ECHUNK)], sv)
            pltpu.sync_copy(ei_hbm.at[pl.ds(E + off, ECHUNK)], dv)
            pltpu.async_copy(s_hbm.at[sv], rv, sem)

        def drain_scatter(sv, dv, rv, sem):
            pltpu.make_async_copy(s_hbm.at[sv], rv, sem).wait()
            pltpu.sync_copy(rv, acc_sh.at[dv], add=True)

        prefetch(0, *bufs[0])
        prefetch(1, *bufs[1])

        @pl.loop(0, nch - 2)
        def _(k):
            for b in range(2):
                @pl.when(lax.bitwise_and(k, 1) == b)
                def _():
                    sv, dv, rv, sem = bufs[b]
                    drain_scatter(sv, dv, rv, sem)
                    prefetch(k + 2, sv, dv, rv, sem)

        for k in (nch - 2, nch - 1):
            drain_scatter(*bufs[k % 2])

        plsc.subcore_barrier()
        for part in range(3):
            pltpu.sync_copy(acc_sh.at[pl.ds(s * ACC_STRIPE + part * ECHUNK, ECHUNK)], rows0)
            pltpu.sync_copy(
                rows0,
                acc_hbm.at[pl.ds(slot * ACC_PAD + s * ACC_STRIPE + part * ECHUNK, ECHUNK)])
        pltpu.sync_copy(acc_sh.at[pl.ds(s * ACC_STRIPE + 3 * ECHUNK, tail)],
                        rows0.at[pl.ds(0, tail)])
        pltpu.sync_copy(
            rows0.at[pl.ds(0, tail)],
            acc_hbm.at[pl.ds(slot * ACC_PAD + s * ACC_STRIPE + 3 * ECHUNK, tail)])

    @pl.when(c == 0)
    def _():
        one_pass(s0, 0, 0, E)
        one_pass(s1, 1, 0, E)
        one_pass(s4, 4, 0, E // 2)

    @pl.when(c == 1)
    def _():
        one_pass(s2, 2, 0, E)
        one_pass(s3, 3, 0, E)
        one_pass(s4, 5, E // 2, E // 2)


def _aggregate(edge_index, slices):
    return pl.kernel(
        _scat_body,
        out_type=jax.ShapeDtypeStruct((NSLOT * ACC_PAD, SL), jnp.float32),
        mesh=_MESH,
        compiler_params=_CP,
        scratch_types=[
            pltpu.VMEM_SHARED((ACC_PAD, SL), jnp.float32),
            pltpu.VMEM((ECHUNK,), jnp.int32),
            pltpu.VMEM((ECHUNK,), jnp.int32),
            pltpu.VMEM((ECHUNK,), jnp.int32),
            pltpu.VMEM((ECHUNK,), jnp.int32),
            pltpu.VMEM((ECHUNK, SL), jnp.float32),
            pltpu.VMEM((ECHUNK, SL), jnp.float32),
            pltpu.VMEM((ECHUNK, SL), jnp.float32),
            pltpu.SemaphoreType.DMA,
            pltpu.SemaphoreType.DMA,
        ],
    )(edge_index.reshape(2 * E), *slices)


def _segmax_body(pre_hbm, coef_hbm, batch_hbm, part_hbm, acc_v, hm_v, bt_v, coef_v):
    c = lax.axis_index("c")
    s = lax.axis_index("s")
    w = s * NC + c
    pltpu.sync_copy(coef_hbm, coef_v)
    neg = jnp.full((16,), -jnp.inf, jnp.float32)
    io = lax.iota(jnp.int32, 16)

    @pl.loop(0, G * D, step=16)
    def _(i):
        acc_v[pl.ds(i, 16)] = neg

    base = _worker_base(w)

    @pl.loop(0, ROWS_W // ROWCHUNK)
    def _(j):
        row0 = base + j * ROWCHUNK
        pltpu.sync_copy(pre_hbm.at[pl.ds(row0, ROWCHUNK)], hm_v)
        pltpu.sync_copy(batch_hbm.at[pl.ds(row0, ROWCHUNK)], bt_v)

        @pl.loop(0, ROWCHUNK)
        def _(r):
            rs = jnp.full((16,), r, jnp.int32)
            gb = plsc.load_gather(bt_v, [rs]) * D
            for c5 in range(5):
                a = coef_v[0, pl.ds(c5 * 16, 16)]
                b = coef_v[1, pl.ds(c5 * 16, 16)]
                xv = hm_v[r, pl.ds(c5 * 16, 16)]
                val = jnp.maximum(xv * a + b, 0.0)
                idx = gb + (io + c5 * 16)
                cur = plsc.load_gather(acc_v, [idx])
                plsc.store_scatter(acc_v, [idx], jnp.maximum(cur, val))

    pltpu.sync_copy(acc_v, part_hbm.at[pl.ds(w * (G * D), G * D)])


def _segmax(pre2, coef2, batch):
    return pl.kernel(
        _segmax_body,
        out_type=jax.ShapeDtypeStruct((NW * G * D,), jnp.float32),
        mesh=_MESH,
        compiler_params=_CP,
        scratch_types=[
            pltpu.VMEM((G * D,), jnp.float32),
            pltpu.VMEM((ROWCHUNK, D), jnp.float32),
            pltpu.VMEM((ROWCHUNK,), jnp.int32),
            pltpu.VMEM((2, D), jnp.float32),
        ],
    )(pre2, coef2, batch)


TB = 2000
NGRID = N // TB


def _slice_outs():
    return ([pl.BlockSpec((TB, SL), lambda i: (i, 0)) for _ in range(5)],
            [jax.ShapeDtypeStruct((N, SL), jnp.float32) for _ in range(5)])


def _prep1_body(h0_ref, degp_ref, w_ref, *out_refs):
    deg = degp_ref[0] + degp_ref[1] + 1.0
    dv = lax.rsqrt(deg)
    hw = jnp.dot(h0_ref[...], w_ref[...], preferred_element_type=jnp.float32)
    sc = dv * hw
    for k in range(5):
        out_refs[k][...] = sc[:, k * SL:(k + 1) * SL]
    out_refs[5][...] = dv


def _prep1(h0, degp, W1):
    sspecs, sshapes = _slice_outs()
    return pl.pallas_call(
        _prep1_body,
        grid=(NGRID,),
        in_specs=[
            pl.BlockSpec((TB, D), lambda i: (i, 0)),
            pl.BlockSpec((NC, TB, 1), lambda i: (0, i, 0)),
            pl.BlockSpec((D, D), lambda i: (0, 0)),
        ],
        out_specs=sspecs + [pl.BlockSpec((TB, 1), lambda i: (i, 0))],
        out_shape=sshapes + [jax.ShapeDtypeStruct((N, 1), jnp.float32)],
    )(h0, degp.reshape(NC, DEG_PAD, 1), W1)


def _post_body(acc_ref, s0, s1, s2, s3, s4, dinv_ref, b_ref, g_ref, be_ref,
               pre_ref, coef_ref, stat_ref):
    i = pl.program_id(0)
    dv = dinv_ref[...]
    agg = jnp.concatenate(
        [acc_ref[0] + s0[...], acc_ref[1] + s1[...], acc_ref[2] + s2[...],
         acc_ref[3] + s3[...], acc_ref[4] + acc_ref[5] + s4[...]], axis=1)
    pre = dv * agg + b_ref[...]
    pre_ref[...] = pre
    ps = jnp.sum(pre, axis=0)
    pq = jnp.sum(pre * pre, axis=0)

    @pl.when(i == 0)
    def _():
        stat_ref[0, :] = ps
        stat_ref[1, :] = pq

    @pl.when(i > 0)
    def _():
        stat_ref[0, :] += ps
        stat_ref[1, :] += pq

    @pl.when(i == NGRID - 1)
    def _():
        mu = stat_ref[0, :] * (1.0 / N)
        var = stat_ref[1, :] * (1.0 / N) - mu * mu
        a = g_ref[...] * lax.rsqrt(var + 1e-5)
        coef_ref[0, :] = a
        coef_ref[1, :] = be_ref[...] - mu * a


def _post(acc, slices, dinv, b, g, be):
    return pl.pallas_call(
        _post_body,
        grid=(NGRID,),
        in_specs=[pl.BlockSpec((NSLOT, TB, SL), lambda i: (0, i, 0))]
        + [pl.BlockSpec((TB, SL), lambda i: (i, 0)) for _ in range(5)]
        + [
            pl.BlockSpec((TB, 1), lambda i: (i, 0)),
            pl.BlockSpec((D,), lambda i: (0,)),
            pl.BlockSpec((D,), lambda i: (0,)),
            pl.BlockSpec((D,), lambda i: (0,)),
        ],
        out_specs=[
            pl.BlockSpec((TB, D), lambda i: (i, 0)),
            pl.BlockSpec((2, D), lambda i: (0, 0)),
        ],
        out_shape=[
            jax.ShapeDtypeStruct((N, D), jnp.float32),
            jax.ShapeDtypeStruct((2, D), jnp.float32),
        ],
        scratch_shapes=[pltpu.VMEM((2, D), jnp.float32)],
    )(acc, *slices, dinv, b, g, be)


def _prep2_body(pre_ref, coef_ref, w_ref, dinv_ref, *out_refs):
    h1 = jnp.maximum(pre_ref[...] * coef_ref[0, :] + coef_ref[1, :], 0.0)
    hw = jnp.dot(h1, w_ref[...], preferred_element_type=jnp.float32)
    sc = dinv_ref[...] * hw
    for k in range(5):
        out_refs[k][...] = sc[:, k * SL:(k + 1) * SL]


def _prep2(pre1, coef1, W2, dinv):
    sspecs, sshapes = _slice_outs()
    return pl.pallas_call(
        _prep2_body,
        grid=(NGRID,),
        in_specs=[
            pl.BlockSpec((TB, D), lambda i: (i, 0)),
            pl.BlockSpec((2, D), lambda i: (0, 0)),
            pl.BlockSpec((D, D), lambda i: (0, 0)),
            pl.BlockSpec((TB, 1), lambda i: (i, 0)),
        ],
        out_specs=sspecs,
        out_shape=sshapes,
    )(pre1, coef1, W2, dinv)


def _head_body(part_ref, lw_ref, lb_ref, out_ref):
    p = part_ref[...].reshape(NW, G, D)
    pooled = jnp.max(p, axis=0)
    out_ref[...] = jnp.dot(pooled, lw_ref[...],
                           preferred_element_type=jnp.float32) + lb_ref[...]


def _head(part, linW, linb):
    return pl.pallas_call(
        _head_body,
        out_shape=jax.ShapeDtypeStruct((G, 20), jnp.float32),
    )(part, linW, linb)


def kernel(x, edge_index, batch, emb, W1, b1, g1, be1, W2, b2, g2, be2, linW, linb):
    h0 = _emb_lookup(x, emb)
    degp = _degrees(edge_index)
    *sl1, dinv = _prep1(h0, degp, W1)
    acc1 = _aggregate(edge_index, sl1).reshape(NSLOT, ACC_PAD, SL)
    pre1, coef1 = _post(acc1, sl1, dinv, b1, g1, be1)
    sl2 = _prep2(pre1, coef1, W2, dinv)
    acc2 = _aggregate(edge_index, sl2).reshape(NSLOT, ACC_PAD, SL)
    pre2, coef2 = _post(acc2, sl2, dinv, b2, g2, be2)
    part = _segmax(pre2, coef2, batch)
    return _head(part.reshape(NW, G * D), linW, linb)


# speedup vs baseline: 21.0549x; 1.1156x over previous
import jax
import jax.numpy as jnp
from jax import lax
from jax.experimental import pallas as pl
from jax.experimental.pallas import tpu as pltpu
from jax.experimental.pallas import tpu_sc as plsc

N = 50000
NP = 51200
E = 800000
G = 500
D = 80
HX = 40
V = 10000

NC = 2
NS = 16
NW = NC * NS

STRIPE = NP // NS
ECHUNK = 1000
SL = 16
ROWS_W = 1568
ROWCHUNK = 224
PARTLEN = 40064

TB = 2048
NGRID = NP // TB

_MESH = plsc.VectorSubcoreMesh(core_axis_name="c", subcore_axis_name="s")
_CP = pltpu.CompilerParams(needs_layout_passes=False, use_tc_tiling_on_sc=False)

STATS = NC * 96
SLICE_OF = ((0, 1, 4), (2, 3, 4))
STAT_OFF = {0: [(0, 16)], 1: [(32, 48)], 2: [(96, 112)], 3: [(128, 144)],
             4: [(64, 80), (160, 176)]}


def _worker_base(w):
    return (N * w // NW) // 8 * 8


def _rsqrt16(d):
    i = plsc.bitcast(d, jnp.int32)
    i = 0x5F3759DF - lax.shift_right_logical(i, 1)
    y = plsc.bitcast(i, jnp.float32)
    for _ in range(3):
        y = y * (1.5 - 0.5 * d * y * y)
    return y


def _emb_body(x_hbm, emb_hbm, h0_hbm, emb_v, xc_v, out_v):
    c = lax.axis_index("c")
    s = lax.axis_index("s")
    w = s * NC + c
    pltpu.sync_copy(emb_hbm, emb_v)
    base = _worker_base(w)
    io = lax.iota(jnp.int32, 16)
    half = lax.shift_right_logical(io, 1)
    par = lax.bitwise_and(io, 1)
    cols = [half + (c5 * 8) for c5 in range(5)]

    @pl.loop(0, ROWS_W // ROWCHUNK)
    def _(j):
        row0 = base + j * ROWCHUNK
        pltpu.sync_copy(x_hbm.at[pl.ds(row0, ROWCHUNK)], xc_v)

        @pl.loop(0, ROWCHUNK)
        def _(r):
            rs = jnp.full((16,), r, jnp.int32)
            for c5 in range(5):
                xv = plsc.load_gather(xc_v, [rs, cols[c5]])
                val = plsc.load_gather(emb_v, [xv + xv + par])
                out_v[r, pl.ds(c5 * 16, 16)] = val

        pltpu.sync_copy(out_v, h0_hbm.at[pl.ds(row0, ROWCHUNK)])


def _emb_lookup(x, emb):
    return pl.kernel(
        _emb_body,
        out_type=jax.ShapeDtypeStruct((NP, D), jnp.float32),
        mesh=_MESH,
        compiler_params=_CP,
        scratch_types=[
            pltpu.VMEM((2 * V,), jnp.float32),
            pltpu.VMEM((ROWCHUNK, HX), jnp.int32),
            pltpu.VMEM((ROWCHUNK, D), jnp.float32),
        ],
    )(x, emb.reshape(2 * V))


def _deg_body(ei_hbm, degp_hbm, acc_sh, ones_v, idx_v, zb_v):
    c = lax.axis_index("c")
    s = lax.axis_index("s")
    zero16 = jnp.zeros((16,), jnp.float32)
    one16 = jnp.ones((16,), jnp.float32)

    @pl.loop(0, STRIPE, step=16)
    def _(i):
        zb_v[pl.ds(i, 16)] = zero16

    @pl.loop(0, ECHUNK - 8, step=16)
    def _(i):
        ones_v[pl.ds(i, 16)] = one16

    ones_v[pl.ds(ECHUNK - 16, 16)] = one16

    pltpu.sync_copy(zb_v, acc_sh.at[pl.ds(s * STRIPE, STRIPE)])
    plsc.subcore_barrier()

    epw = E // NC // NS

    @pl.loop(0, epw // ECHUNK)
    def _(k):
        off = c * (E // NC) + s * epw + k * ECHUNK
        pltpu.sync_copy(ei_hbm.at[1, pl.ds(off, ECHUNK)], idx_v)
        pltpu.sync_copy(ones_v, acc_sh.at[idx_v], add=True)

    plsc.subcore_barrier()
    pltpu.sync_copy(acc_sh.at[pl.ds(s * STRIPE, STRIPE)], zb_v)
    pltpu.sync_copy(zb_v, degp_hbm.at[pl.ds(c * NP + s * STRIPE, STRIPE)])


def _degrees(edge_index):
    return pl.kernel(
        _deg_body,
        out_type=jax.ShapeDtypeStruct((NC * NP,), jnp.float32),
        mesh=_MESH,
        compiler_params=_CP,
        scratch_types=[
            pltpu.VMEM_SHARED((NP,), jnp.float32),
            pltpu.VMEM((ECHUNK,), jnp.float32),
            pltpu.VMEM((ECHUNK,), jnp.int32),
            pltpu.VMEM((STRIPE,), jnp.float32),
        ],
    )(edge_index)


def _scat_body(ei_hbm, s0, s1, s2, s3, s4, degp_hbm, b_hbm, pidx_hbm,
               pre_hbm, stats_hbm, acc_sh, stats_sh,
               src0, dst0, src1, dst1, rows0, rows1, zb_v,
               dv_v, d0_v, d1_v, b_v, st_v, pc_v, semA, semB):
    c = lax.axis_index("c")
    s = lax.axis_index("s")
    zero16 = jnp.zeros((16,), jnp.float32)

    @pl.loop(0, ECHUNK)
    def _(r):
        zb_v[r, :] = zero16

    pltpu.sync_copy(degp_hbm.at[pl.ds(s * STRIPE, STRIPE)], d0_v)
    pltpu.sync_copy(degp_hbm.at[pl.ds(NP + s * STRIPE, STRIPE)], d1_v)
    pltpu.sync_copy(b_hbm, b_v)
    pltpu.sync_copy(pidx_hbm, pc_v)

    @pl.loop(0, STRIPE, step=16)
    def _(i):
        d = d0_v[pl.ds(i, 16)] + d1_v[pl.ds(i, 16)] + 1.0
        dv_v[pl.ds(i, 16)] = _rsqrt16(d)

    st_v[0, pl.ds(0, 16)] = zero16
    st_v[0, pl.ds(16, 16)] = zero16

    @pl.when(s == 0)
    def _():
        for p in range(3):
            pltpu.sync_copy(st_v, stats_sh.at[pl.ds(p, 1)])

    for part in range(3):
        pltpu.sync_copy(zb_v, acc_sh.at[pl.ds(s * STRIPE + part * ECHUNK, ECHUNK)])
    pltpu.sync_copy(zb_v.at[pl.ds(0, STRIPE - 3 * ECHUNK)],
                    acc_sh.at[pl.ds(s * STRIPE + 3 * ECHUNK, STRIPE - 3 * ECHUNK)])
    plsc.subcore_barrier()

    bufs = ((src0, dst0, rows0, semA), (src1, dst1, rows1, semB))

    def one_pass(s_hbm, kslice, pass_id, elo, nedge, last):
        epw = nedge // NS
        nch = epw // ECHUNK
        base = elo + s * epw

        def prefetch(k, sv, dv, rv, sem):
            off = base + k * ECHUNK
            pltpu.sync_copy(ei_hbm.at[0, pl.ds(off, ECHUNK)], sv)
            pltpu.sync_copy(ei_hbm.at[1, pl.ds(off, ECHUNK)], dv)
            pltpu.async_copy(s_hbm.at[sv], rv, sem)

        def drain_scatter(sv, dv, rv, sem):
            pltpu.make_async_copy(s_hbm.at[sv], rv, sem).wait()
            pltpu.sync_copy(rv, acc_sh.at[dv], add=True)

        prefetch(0, *bufs[0])
        prefetch(1, *bufs[1])

        @pl.loop(0, nch - 2)
        def _(k):
            for b in range(2):
                @pl.when(lax.bitwise_and(k, 1) == b)
                def _():
                    sv, dv, rv, sem = bufs[b]
                    drain_scatter(sv, dv, rv, sem)
                    prefetch(k + 2, sv, dv, rv, sem)

        for k in (nch - 2, nch - 1):
            drain_scatter(*bufs[k % 2])

        plsc.subcore_barrier()

        b16 = b_v[pl.ds(kslice * 16, 16)]
        tot_s, tot_q = zero16, zero16
        for part, cnt in ((0, ECHUNK), (1, ECHUNK), (2, ECHUNK), (3, STRIPE - 3 * ECHUNK)):
            row0 = s * STRIPE + part * ECHUNK
            pltpu.sync_copy(acc_sh.at[pl.ds(row0, cnt)], rows0.at[pl.ds(0, cnt)])
            pltpu.sync_copy(s_hbm.at[pl.ds(row0, cnt)], rows1.at[pl.ds(0, cnt)])

            @pl.loop(0, cnt, init_carry=(tot_s, tot_q))
            def postloop(r, carry):
                ts, tq = carry
                a = rows0[r, :]
                sv = rows1[r, :]
                dloc = part * ECHUNK + r
                dvs = plsc.load_gather(dv_v, [jnp.full((16,), dloc, jnp.int32)])
                pre_r = dvs * (a + sv) + b16
                rows0[r, :] = pre_r
                m = jnp.where(row0 + r < N, 1.0, 0.0)
                pm = pre_r * m
                return ts + pm, tq + pm * pm

            tot_s, tot_q = postloop
            pltpu.sync_copy(rows0.at[pl.ds(0, cnt)],
                            pre_hbm.at[pl.ds(row0, cnt), pl.ds(kslice * 16, 16)])

        st_v[0, pl.ds(0, 16)] = tot_s
        st_v[0, pl.ds(16, 16)] = tot_q
        pltpu.sync_copy(st_v, stats_sh.at[pc_v.at[pl.ds(pass_id * 8, 1)]], add=True)
        if not last:
            for part in range(3):
                pltpu.sync_copy(zb_v, acc_sh.at[pl.ds(s * STRIPE + part * ECHUNK, ECHUNK)])
            pltpu.sync_copy(zb_v.at[pl.ds(0, STRIPE - 3 * ECHUNK)],
                            acc_sh.at[pl.ds(s * STRIPE + 3 * ECHUNK, STRIPE - 3 * ECHUNK)])
            plsc.subcore_barrier()

    @pl.when(c == 0)
    def _():
        one_pass(s0, 0, 0, 0, E, False)
        one_pass(s1, 1, 1, 0, E, False)
        one_pass(s4, 4, 2, 0, E // 2, True)

    @pl.when(c == 1)
    def _():
        one_pass(s2, 2, 0, 0, E, False)
        one_pass(s3, 3, 1, 0, E, False)
        one_pass(s4, 4, 2, E // 2, E // 2, True)

    plsc.subcore_barrier()

    @pl.when(s == 0)
    def _():
        for p in range(3):
            pltpu.sync_copy(stats_sh.at[pl.ds(p, 1)], st_v)
            pltpu.sync_copy(st_v, stats_hbm.at[pl.ds(c * 3 + p, 1)])


def _aggregate(edge_index, slices, degp, b, pidx):
    return pl.kernel(
        _scat_body,
        out_type=(jax.ShapeDtypeStruct((NP, D), jnp.float32),
                  jax.ShapeDtypeStruct((NC * 3, 32), jnp.float32)),
        mesh=_MESH,
        compiler_params=_CP,
        scratch_types=[
            pltpu.VMEM_SHARED((NP, SL), jnp.float32),
            pltpu.VMEM_SHARED((3, 32), jnp.float32),
            pltpu.VMEM((ECHUNK,), jnp.int32),
            pltpu.VMEM((ECHUNK,), jnp.int32),
            pltpu.VMEM((ECHUNK,), jnp.int32),
            pltpu.VMEM((ECHUNK,), jnp.int32),
            pltpu.VMEM((ECHUNK, SL), jnp.float32),
            pltpu.VMEM((ECHUNK, SL), jnp.float32),
            pltpu.VMEM((ECHUNK, SL), jnp.float32),
            pltpu.VMEM((STRIPE,), jnp.float32),
            pltpu.VMEM((STRIPE,), jnp.float32),
            pltpu.VMEM((STRIPE,), jnp.float32),
            pltpu.VMEM((D,), jnp.float32),
            pltpu.VMEM((1, 32), jnp.float32),
            pltpu.VMEM((24,), jnp.int32),
            pltpu.SemaphoreType.DMA,
            pltpu.SemaphoreType.DMA,
        ],
    )(edge_index, *slices, degp, b, pidx)


def _segmax_body(pre_hbm, stats_hbm, g_hbm, be_hbm, batch_hbm, part_hbm,
                 acc_v, hm_v, bt_v, coef_v, st_v, g_v, be_v):
    c = lax.axis_index("c")
    s = lax.axis_index("s")
    w = s * NC + c
    pltpu.sync_copy(stats_hbm, st_v)
    pltpu.sync_copy(g_hbm, g_v)
    pltpu.sync_copy(be_hbm, be_v)
    for k in range(5):
        ts = zero = jnp.zeros((16,), jnp.float32)
        tq = zero
        for (so, qo) in STAT_OFF[k]:
            ts = ts + st_v[pl.ds(so, 16)]
            tq = tq + st_v[pl.ds(qo, 16)]
        mu = ts * (1.0 / N)
        var = tq * (1.0 / N) - mu * mu
        a = g_v[pl.ds(k * 16, 16)] * _rsqrt16(var + 1e-5)
        coef_v[0, pl.ds(k * 16, 16)] = a
        coef_v[1, pl.ds(k * 16, 16)] = be_v[pl.ds(k * 16, 16)] - mu * a

    neg = jnp.full((16,), -jnp.inf, jnp.float32)
    io = lax.iota(jnp.int32, 16)

    @pl.loop(0, PARTLEN, step=16)
    def _(i):
        acc_v[pl.ds(i, 16)] = neg

    base = _worker_base(w)

    @pl.loop(0, ROWS_W // ROWCHUNK)
    def _(j):
        row0 = base + j * ROWCHUNK
        pltpu.sync_copy(pre_hbm.at[pl.ds(row0, ROWCHUNK)], hm_v)
        pltpu.sync_copy(batch_hbm.at[pl.ds(row0, ROWCHUNK)], bt_v)

        @pl.loop(0, ROWCHUNK)
        def _(r):
            rs = jnp.full((16,), r, jnp.int32)
            gb = plsc.load_gather(bt_v, [rs]) * D
            for c5 in range(5):
                a = coef_v[0, pl.ds(c5 * 16, 16)]
                b = coef_v[1, pl.ds(c5 * 16, 16)]
                xv = hm_v[r, pl.ds(c5 * 16, 16)]
                val = jnp.maximum(xv * a + b, 0.0)
                idx = gb + (io + c5 * 16)
                cur = plsc.load_gather(acc_v, [idx])
                plsc.store_scatter(acc_v, [idx], jnp.maximum(cur, val))

    pltpu.sync_copy(acc_v, part_hbm.at[pl.ds(w * PARTLEN, PARTLEN)])


def _segmax(pre2, stats2, g, be, batch):
    return pl.kernel(
        _segmax_body,
        out_type=jax.ShapeDtypeStruct((NW * PARTLEN,), jnp.float32),
        mesh=_MESH,
        compiler_params=_CP,
        scratch_types=[
            pltpu.VMEM((PARTLEN,), jnp.float32),
            pltpu.VMEM((ROWCHUNK, D), jnp.float32),
            pltpu.VMEM((ROWCHUNK,), jnp.int32),
            pltpu.VMEM((2, D), jnp.float32),
            pltpu.VMEM((STATS,), jnp.float32),
            pltpu.VMEM((D,), jnp.float32),
            pltpu.VMEM((D,), jnp.float32),
        ],
    )(pre2, stats2, g, be, batch)


def _slice_outs():
    return ([pl.BlockSpec((TB, SL), lambda i: (i, 0)) for _ in range(5)],
            [jax.ShapeDtypeStruct((NP, SL), jnp.float32) for _ in range(5)])


def _prep1_body(h0_ref, degp_ref, w_ref, *out_refs):
    h0 = h0_ref[...]
    deg = degp_ref[0] + degp_ref[1] + 1.0
    dv = lax.rsqrt(deg)
    hw = jnp.dot(h0, w_ref[...], preferred_element_type=jnp.float32)
    sc = dv * hw
    for k in range(5):
        out_refs[k][...] = sc[:, k * SL:(k + 1) * SL]
    out_refs[5][...] = dv


def _prep1(h0, degp, W1):
    sspecs, sshapes = _slice_outs()
    return pl.pallas_call(
        _prep1_body,
        grid=(NGRID,),
        in_specs=[
            pl.BlockSpec((TB, D), lambda i: (i, 0)),
            pl.BlockSpec((NC, TB, 1), lambda i: (0, i, 0)),
            pl.BlockSpec((D, D), lambda i: (0, 0)),
        ],
        out_specs=sspecs + [pl.BlockSpec((TB, 1), lambda i: (i, 0))],
        out_shape=sshapes + [jax.ShapeDtypeStruct((NP, 1), jnp.float32)],
    )(h0, degp.reshape(NC, NP, 1), W1)


def _prep2_body(pre_ref, st_ref, g_ref, be_ref, w_ref, dinv_ref, *out_refs):
    st = st_ref[...]
    parts = []
    for k in range(5):
        ts = sum(lax.slice(st, (so,), (so + 16,)) for so, _ in STAT_OFF[k])
        tq = sum(lax.slice(st, (qo,), (qo + 16,)) for _, qo in STAT_OFF[k])
        parts.append((ts, tq))
    mu = jnp.concatenate([p[0] for p in parts]) * (1.0 / N)
    var = jnp.concatenate([p[1] for p in parts]) * (1.0 / N) - mu * mu
    a = g_ref[...] * lax.rsqrt(var + 1e-5)
    b = be_ref[...] - mu * a
    h1 = jnp.maximum(pre_ref[...] * a + b, 0.0)
    hw = jnp.dot(h1, w_ref[...], preferred_element_type=jnp.float32)
    sc = dinv_ref[...] * hw
    for k in range(5):
        out_refs[k][...] = sc[:, k * SL:(k + 1) * SL]


def _prep2(pre1, stats1, g, be, W2, dinv):
    sspecs, sshapes = _slice_outs()
    return pl.pallas_call(
        _prep2_body,
        grid=(NGRID,),
        in_specs=[
            pl.BlockSpec((TB, D), lambda i: (i, 0)),
            pl.BlockSpec((STATS,), lambda i: (0,)),
            pl.BlockSpec((D,), lambda i: (0,)),
            pl.BlockSpec((D,), lambda i: (0,)),
            pl.BlockSpec((D, D), lambda i: (0, 0)),
            pl.BlockSpec((TB, 1), lambda i: (i, 0)),
        ],
        out_specs=sspecs,
        out_shape=sshapes,
    )(pre1, stats1, g, be, W2, dinv)


def _head_body(part_ref, lw_ref, lb_ref, out_ref):
    p = part_ref[...][:, :G * D].reshape(NW, G, D)
    pooled = jnp.max(p, axis=0)
    out_ref[...] = jnp.dot(pooled, lw_ref[...],
                           preferred_element_type=jnp.float32) + lb_ref[...]


def _head(part, linW, linb):
    return pl.pallas_call(
        _head_body,
        out_shape=jax.ShapeDtypeStruct((G, 20), jnp.float32),
    )(part.reshape(NW, PARTLEN), linW, linb)


def kernel(x, edge_index, batch, emb, W1, b1, g1, be1, W2, b2, g2, be2, linW, linb):
    pidx = (jnp.arange(24, dtype=jnp.int32) // 8).astype(jnp.int32)
    h0 = _emb_lookup(x, emb)
    degp = _degrees(edge_index)
    *sl1, dinv = _prep1(h0, degp, W1)
    pre1, stats1 = _aggregate(edge_index, sl1, degp, b1, pidx)
    sl2 = _prep2(pre1, stats1.reshape(STATS), g1, be1, W2, dinv)
    pre2, stats2 = _aggregate(edge_index, sl2, degp, b2, pidx)
    part = _segmax(pre2, stats2.reshape(STATS), g2, be2, batch)
    return _head(part, linW, linb)
